# baseline (speedup 1.0000x reference)
import dataclasses
import functools

import jax
import jax.numpy as jnp
from jax import lax
from jax.experimental import pallas as pl
from jax.experimental.pallas import tpu as pltpu
from jax.experimental.pallas import tpu_sc as plsc

N = 10000
D = 128
E = 320000
NC = 2
NS = 16
NW = NC * NS
CHUNK = 128
ACC_ROWS = 10240
GARBAGE = 10200

E_PAD = 327680
P_PAD = 16384
CPW_EDGE = E_PAD // (NW * CHUNK)
CPW_POOL = P_PAD // (NW * CHUNK)
STRIPE = ACC_ROWS // NS

_MESH = plsc.VectorSubcoreMesh(core_axis_name="c", subcore_axis_name="s")


def _fill(ref, value):
    @pl.loop(0, CHUNK)
    def _(r):
        @pl.loop(0, 8)
        def _(cc):
            ref[r, pl.ds(cc * 16, 16)] = jnp.full((16,), value, jnp.float32)


def _zero_acc(zeros_v, acc_sh, sid):
    @pl.loop(0, STRIPE // CHUNK)
    def _(t):
        pltpu.sync_copy(zeros_v, acc_sh.at[pl.ds(sid * STRIPE + t * CHUNK, CHUNK)])


def _make_sc_agg(cpw, nhalves):
    hc = cpw // nhalves

    @functools.partial(
        pl.kernel,
        out_type=jax.ShapeDtypeStruct((NC, ACC_ROWS, 128), jnp.float32),
        mesh=_MESH,
        scratch_types=[
            pltpu.VMEM((hc, CHUNK), jnp.int32),
            pltpu.VMEM((hc, CHUNK), jnp.int32),
            pltpu.VMEM((CHUNK, 128), jnp.float32),
            pltpu.VMEM((CHUNK, 128), jnp.float32),
            pltpu.SemaphoreType.DMA,
            pltpu.SemaphoreType.DMA,
            pltpu.VMEM_SHARED((ACC_ROWS, 128), jnp.float32),
        ],
    )
    def k(vals_hbm, src_hbm, dst_hbm, out_hbm,
          src_v, dst_v, ra, rb, sa, sb, acc_sh):
        core = lax.axis_index("c")
        sid = lax.axis_index("s")
        w = core * NS + sid

        _fill(ra, 0.0)
        _zero_acc(ra, acc_sh, sid)
        plsc.subcore_barrier()

        for h in range(nhalves):
            if nhalves == 1:
                pltpu.sync_copy(src_hbm.at[w], src_v)
                pltpu.sync_copy(dst_hbm.at[w], dst_v)
            else:
                pltpu.sync_copy(src_hbm.at[w, pl.ds(h * hc, hc)], src_v)
                pltpu.sync_copy(dst_hbm.at[w, pl.ds(h * hc, hc)], dst_v)

            pltpu.async_copy(vals_hbm.at[src_v.at[0]], ra, sa)

            @pl.loop(0, (hc - 2) // 2)
            def _(t):
                j0 = 2 * t
                pltpu.async_copy(vals_hbm.at[src_v.at[j0 + 1]], rb, sb)
                pltpu.make_async_copy(vals_hbm.at[src_v.at[j0]], ra, sa).wait()
                pltpu.sync_copy(ra, acc_sh.at[dst_v.at[j0]], add=True)
                pltpu.async_copy(vals_hbm.at[src_v.at[j0 + 2]], ra, sa)
                pltpu.make_async_copy(vals_hbm.at[src_v.at[j0 + 1]], rb, sb).wait()
                pltpu.sync_copy(rb, acc_sh.at[dst_v.at[j0 + 1]], add=True)

            pltpu.async_copy(vals_hbm.at[src_v.at[hc - 1]], rb, sb)
            pltpu.make_async_copy(vals_hbm.at[src_v.at[hc - 2]], ra, sa).wait()
            pltpu.sync_copy(ra, acc_sh.at[dst_v.at[hc - 2]], add=True)
            pltpu.make_async_copy(vals_hbm.at[src_v.at[hc - 1]], rb, sb).wait()
            pltpu.sync_copy(rb, acc_sh.at[dst_v.at[hc - 1]], add=True)

        plsc.subcore_barrier()
        pltpu.sync_copy(acc_sh.at[pl.ds(sid * STRIPE, STRIPE)],
                        out_hbm.at[core, pl.ds(sid * STRIPE, STRIPE)])

    return k


def _make_sc_count():
    cp = pltpu.CompilerParams()
    if "needs_layout_passes" in pltpu.CompilerParams.__dataclass_fields__:
        cp = dataclasses.replace(cp, needs_layout_passes=False)

    @functools.partial(
        pl.kernel,
        out_type=jax.ShapeDtypeStruct((NC, NS, 2, ACC_ROWS), jnp.float32),
        mesh=_MESH,
        compiler_params=cp,
        scratch_types=[
            pltpu.VMEM((CPW_EDGE, CHUNK), jnp.int32),
            pltpu.VMEM((CPW_POOL, CHUNK), jnp.int32),
            pltpu.VMEM((ACC_ROWS,), jnp.float32),
            pltpu.VMEM((ACC_ROWS,), jnp.float32),
        ],
    )
    def k(eidx_hbm, pidx_hbm, out_hbm, eidx_v, pidx_v, ehist, phist):
        core = lax.axis_index("c")
        sid = lax.axis_index("s")
        w = core * NS + sid

        pltpu.sync_copy(eidx_hbm.at[w], eidx_v)
        pltpu.sync_copy(pidx_hbm.at[w], pidx_v)

        zeros16 = jnp.zeros((16,), jnp.float32)

        @pl.loop(0, ACC_ROWS // 16)
        def _(i):
            ehist[pl.ds(i * 16, 16)] = zeros16
            phist[pl.ds(i * 16, 16)] = zeros16

        ones16 = jnp.ones((16,), jnp.float32)

        @pl.loop(0, CPW_EDGE)
        def _(j):
            @pl.loop(0, CHUNK // 16)
            def _(g):
                idx = eidx_v[j, pl.ds(g * 16, 16)]
                plsc.addupdate_scatter(ehist, [idx], ones16)

        @pl.loop(0, CPW_POOL)
        def _(j):
            @pl.loop(0, CHUNK // 16)
            def _(g):
                idx = pidx_v[j, pl.ds(g * 16, 16)]
                plsc.addupdate_scatter(phist, [idx], ones16)

        pltpu.sync_copy(ehist, out_hbm.at[core, sid, 0])
        pltpu.sync_copy(phist, out_hbm.at[core, sid, 1])

    return k


TPW = ACC_ROWS // NW
PTILE = 64
NT = TPW // PTILE


def _make_sc_pool():
    @functools.partial(
        pl.kernel,
        out_type=jax.ShapeDtypeStruct((NC, ACC_ROWS, 128), jnp.float32),
        mesh=_MESH,
        scratch_types=[
            pltpu.VMEM((NT, PTILE), jnp.int32),
            pltpu.VMEM((PTILE, 128), jnp.float32),
            pltpu.VMEM((PTILE, 128), jnp.float32),
            pltpu.SemaphoreType.DMA,
            pltpu.SemaphoreType.DMA,
            pltpu.VMEM_SHARED((ACC_ROWS, 128), jnp.float32),
        ],
    )
    def k(h2_hbm, didx_hbm, out_hbm, idx_v, ra, rb, sa, sb, acc_sh):
        core = lax.axis_index("c")
        sid = lax.axis_index("s")
        w = core * NS + sid
        base = w * TPW

        @pl.loop(0, PTILE)
        def _(r):
            @pl.loop(0, 8)
            def _(cc):
                ra[r, pl.ds(cc * 16, 16)] = jnp.zeros((16,), jnp.float32)

        @pl.loop(0, STRIPE // PTILE)
        def _(t):
            pltpu.sync_copy(ra, acc_sh.at[pl.ds(sid * STRIPE + t * PTILE, PTILE)])

        pltpu.sync_copy(didx_hbm.at[w], idx_v)
        plsc.subcore_barrier()

        def load(t, buf, sem):
            return pltpu.async_copy(
                h2_hbm.at[pl.ds(base + t * PTILE, PTILE)], buf, sem)

        def wait(t, buf, sem):
            pltpu.make_async_copy(
                h2_hbm.at[pl.ds(base + t * PTILE, PTILE)], buf, sem).wait()

        load(0, ra, sa)

        @pl.loop(0, (NT - 1) // 2)
        def _(t):
            j0 = 2 * t
            load(j0 + 1, rb, sb)
            wait(j0, ra, sa)
            pltpu.sync_copy(ra, acc_sh.at[idx_v.at[j0]], add=True)
            load(j0 + 2, ra, sa)
            wait(j0 + 1, rb, sb)
            pltpu.sync_copy(rb, acc_sh.at[idx_v.at[j0 + 1]], add=True)

        wait(NT - 1, ra, sa)
        pltpu.sync_copy(ra, acc_sh.at[idx_v.at[NT - 1]], add=True)

        plsc.subcore_barrier()
        pltpu.sync_copy(acc_sh.at[pl.ds(sid * STRIPE, STRIPE)],
                        out_hbm.at[core, pl.ds(sid * STRIPE, STRIPE)])

    return k


_SC_AGG_EDGE = _make_sc_agg(CPW_EDGE, 2)
_SC_POOL = _make_sc_pool()
_SC_COUNT = _make_sc_count()


def _mm_scale_body(cnt_ref, x_ref, w_ref, y_ref, dinv_ref):
    deg = jnp.sum(cnt_ref[:, 0, :N], axis=0)[:, None] + 1.0
    dinv = lax.rsqrt(jnp.maximum(deg, 1.0))
    dinv_ref[...] = dinv
    y_ref[...] = dinv * jnp.dot(x_ref[...].astype(jnp.bfloat16),
                                w_ref[...].astype(jnp.bfloat16),
                                preferred_element_type=jnp.float32)


def _mid_body(p_ref, y_ref, dinv_ref, b_ref, w_ref, o_ref):
    tot = p_ref[0, :N, :] + p_ref[1, :N, :] + y_ref[...]
    h = jnp.maximum(dinv_ref[...] * tot + b_ref[...], 0.0)
    o_ref[...] = dinv_ref[...] * jnp.dot(h.astype(jnp.bfloat16),
                                         w_ref[...].astype(jnp.bfloat16),
                                         preferred_element_type=jnp.float32)


def _final_body(p_ref, y_ref, dinv_ref, b_ref, o_ref):
    tot = p_ref[0, :N, :] + p_ref[1, :N, :] + y_ref[...]
    h2 = jnp.maximum(dinv_ref[...] * tot + b_ref[...], 0.0)
    o_ref[...] = jnp.concatenate(
        [h2, jnp.zeros((ACC_ROWS - N, D), jnp.float32)], axis=0)


def _div_body(sp_ref, cp_ref, z_ref):
    s = sp_ref[0, :N, :] + sp_ref[1, :N, :]
    cnt = jnp.sum(cp_ref[:, 1, :N], axis=0)[:, None]
    z_ref[...] = s / jnp.maximum(cnt, 1.0)


def _f32(shape):
    return jax.ShapeDtypeStruct(shape, jnp.float32)


def kernel(x, train_pos_edge_index, dict_node, W1, b1, W2, b2):
    src = train_pos_edge_index[0].astype(jnp.int32)
    dst = train_pos_edge_index[1].astype(jnp.int32)

    pad_e = jnp.arange(E_PAD - E, dtype=jnp.int32)
    src_p = jnp.concatenate(
        [src, pad_e % N]).reshape(NW, CPW_EDGE, CHUNK)
    dst_p = jnp.concatenate(
        [dst, N + pad_e % (ACC_ROWS - N)]).reshape(NW, CPW_EDGE, CHUNK)

    pad_p = jnp.arange(P_PAD - N, dtype=jnp.int32)
    pool_dst = jnp.concatenate(
        [dict_node.astype(jnp.int32),
         N + pad_p % (ACC_ROWS - N)]).reshape(NW, CPW_POOL, CHUNK)
    dpool = jnp.concatenate(
        [dict_node.astype(jnp.int32),
         N + jnp.arange(ACC_ROWS - N, dtype=jnp.int32)]).reshape(NW, NT, PTILE)

    counts = _SC_COUNT(dst_p, pool_dst).reshape(NW, 2, ACC_ROWS)
    y1, dinv = pl.pallas_call(
        _mm_scale_body,
        out_shape=(_f32((N, D)), _f32((N, 1))))(counts, x, W1)

    p1 = _SC_AGG_EDGE(y1, src_p, dst_p)
    y2 = pl.pallas_call(_mid_body, out_shape=_f32((N, D)))(
        p1, y1, dinv, b1.reshape(1, D), W2)

    p2 = _SC_AGG_EDGE(y2, src_p, dst_p)
    h2 = pl.pallas_call(_final_body, out_shape=_f32((ACC_ROWS, D)))(
        p2, y2, dinv, b2.reshape(1, D))

    sp = _SC_POOL(h2, dpool)
    z = pl.pallas_call(_div_body, out_shape=_f32((N, D)))(sp, counts)
    return z


# speedup vs baseline: 1.0031x; 1.0031x over previous
import dataclasses
import functools

import jax
import jax.numpy as jnp
from jax import lax
from jax.experimental import pallas as pl
from jax.experimental.pallas import tpu as pltpu
from jax.experimental.pallas import tpu_sc as plsc

N = 10000
D = 128
E = 320000
NC = 2
NS = 16
NW = NC * NS
CHUNK = 128
ACC_ROWS = 10240

E_PAD = 327680
P_PAD = 16384
CPW_EDGE = E_PAD // (NW * CHUNK)
CPW_POOL = P_PAD // (NW * CHUNK)
STRIPE = ACC_ROWS // NS

_MESH = plsc.VectorSubcoreMesh(core_axis_name="c", subcore_axis_name="s")


def _fill(ref, value):
    @pl.loop(0, CHUNK)
    def _(r):
        @pl.loop(0, 8)
        def _(cc):
            ref[r, pl.ds(cc * 16, 16)] = jnp.full((16,), value, jnp.float32)


def _zero_acc(zeros_v, acc_sh, sid):
    @pl.loop(0, STRIPE // CHUNK)
    def _(t):
        pltpu.sync_copy(zeros_v, acc_sh.at[pl.ds(sid * STRIPE + t * CHUNK, CHUNK)])


def _make_sc_agg(cpw, nhalves):
    hc = cpw // nhalves

    @functools.partial(
        pl.kernel,
        out_type=jax.ShapeDtypeStruct((NC, ACC_ROWS, 128), jnp.float32),
        mesh=_MESH,
        scratch_types=[
            pltpu.VMEM((hc, CHUNK), jnp.int32),
            pltpu.VMEM((hc, CHUNK), jnp.int32),
            pltpu.VMEM((CHUNK, 128), jnp.float32),
            pltpu.VMEM((CHUNK, 128), jnp.float32),
            pltpu.SemaphoreType.DMA,
            pltpu.SemaphoreType.DMA,
            pltpu.VMEM_SHARED((ACC_ROWS, 128), jnp.float32),
        ],
    )
    def k(vals_hbm, src_hbm, dst_hbm, out_hbm,
          src_v, dst_v, ra, rb, sa, sb, acc_sh):
        core = lax.axis_index("c")
        sid = lax.axis_index("s")
        w = core * NS + sid

        _fill(ra, 0.0)
        _zero_acc(ra, acc_sh, sid)
        plsc.subcore_barrier()

        for h in range(nhalves):
            if nhalves == 1:
                pltpu.sync_copy(src_hbm.at[w], src_v)
                pltpu.sync_copy(dst_hbm.at[w], dst_v)
            else:
                pltpu.sync_copy(src_hbm.at[w, pl.ds(h * hc, hc)], src_v)
                pltpu.sync_copy(dst_hbm.at[w, pl.ds(h * hc, hc)], dst_v)

            pltpu.async_copy(vals_hbm.at[src_v.at[0]], ra, sa)

            @pl.loop(0, (hc - 2) // 2)
            def _(t):
                j0 = 2 * t
                pltpu.async_copy(vals_hbm.at[src_v.at[j0 + 1]], rb, sb)
                pltpu.make_async_copy(vals_hbm.at[src_v.at[j0]], ra, sa).wait()
                pltpu.sync_copy(ra, acc_sh.at[dst_v.at[j0]], add=True)
                pltpu.async_copy(vals_hbm.at[src_v.at[j0 + 2]], ra, sa)
                pltpu.make_async_copy(vals_hbm.at[src_v.at[j0 + 1]], rb, sb).wait()
                pltpu.sync_copy(rb, acc_sh.at[dst_v.at[j0 + 1]], add=True)

            pltpu.async_copy(vals_hbm.at[src_v.at[hc - 1]], rb, sb)
            pltpu.make_async_copy(vals_hbm.at[src_v.at[hc - 2]], ra, sa).wait()
            pltpu.sync_copy(ra, acc_sh.at[dst_v.at[hc - 2]], add=True)
            pltpu.make_async_copy(vals_hbm.at[src_v.at[hc - 1]], rb, sb).wait()
            pltpu.sync_copy(rb, acc_sh.at[dst_v.at[hc - 1]], add=True)

        plsc.subcore_barrier()
        pltpu.sync_copy(acc_sh.at[pl.ds(sid * STRIPE, STRIPE)],
                        out_hbm.at[core, pl.ds(sid * STRIPE, STRIPE)])

    return k


def _make_sc_count():
    cp = pltpu.CompilerParams()
    if "needs_layout_passes" in pltpu.CompilerParams.__dataclass_fields__:
        cp = dataclasses.replace(cp, needs_layout_passes=False)

    @functools.partial(
        pl.kernel,
        out_type=jax.ShapeDtypeStruct((NC, NS, 2, ACC_ROWS), jnp.float32),
        mesh=_MESH,
        compiler_params=cp,
        scratch_types=[
            pltpu.VMEM((CPW_EDGE, CHUNK), jnp.int32),
            pltpu.VMEM((CPW_POOL, CHUNK), jnp.int32),
            pltpu.VMEM((ACC_ROWS,), jnp.float32),
            pltpu.VMEM((ACC_ROWS,), jnp.float32),
        ],
    )
    def k(eidx_hbm, pidx_hbm, out_hbm, eidx_v, pidx_v, ehist, phist):
        core = lax.axis_index("c")
        sid = lax.axis_index("s")
        w = core * NS + sid

        pltpu.sync_copy(eidx_hbm.at[w], eidx_v)
        pltpu.sync_copy(pidx_hbm.at[w], pidx_v)

        zeros16 = jnp.zeros((16,), jnp.float32)

        @pl.loop(0, ACC_ROWS // 16)
        def _(i):
            ehist[pl.ds(i * 16, 16)] = zeros16
            phist[pl.ds(i * 16, 16)] = zeros16

        ones16 = jnp.ones((16,), jnp.float32)

        @pl.loop(0, CPW_EDGE)
        def _(j):
            @pl.loop(0, CHUNK // 16)
            def _(g):
                idx = eidx_v[j, pl.ds(g * 16, 16)]
                plsc.addupdate_scatter(ehist, [idx], ones16)

        @pl.loop(0, CPW_POOL)
        def _(j):
            @pl.loop(0, CHUNK // 16)
            def _(g):
                idx = pidx_v[j, pl.ds(g * 16, 16)]
                plsc.addupdate_scatter(phist, [idx], ones16)

        pltpu.sync_copy(ehist, out_hbm.at[core, sid, 0])
        pltpu.sync_copy(phist, out_hbm.at[core, sid, 1])

    return k


TPW = ACC_ROWS // NW
PTILE = 64
NT = TPW // PTILE


def _make_sc_pool():
    @functools.partial(
        pl.kernel,
        out_type=jax.ShapeDtypeStruct((NC, ACC_ROWS, 128), jnp.float32),
        mesh=_MESH,
        scratch_types=[
            pltpu.VMEM((NT, PTILE), jnp.int32),
            pltpu.VMEM((PTILE, 128), jnp.float32),
            pltpu.VMEM((PTILE, 128), jnp.float32),
            pltpu.SemaphoreType.DMA,
            pltpu.SemaphoreType.DMA,
            pltpu.VMEM_SHARED((ACC_ROWS, 128), jnp.float32),
        ],
    )
    def k(h2_hbm, didx_hbm, out_hbm, idx_v, ra, rb, sa, sb, acc_sh):
        core = lax.axis_index("c")
        sid = lax.axis_index("s")
        w = core * NS + sid
        base = w * TPW

        @pl.loop(0, PTILE)
        def _(r):
            @pl.loop(0, 8)
            def _(cc):
                ra[r, pl.ds(cc * 16, 16)] = jnp.zeros((16,), jnp.float32)

        @pl.loop(0, STRIPE // PTILE)
        def _(t):
            pltpu.sync_copy(ra, acc_sh.at[pl.ds(sid * STRIPE + t * PTILE, PTILE)])

        pltpu.sync_copy(didx_hbm.at[w], idx_v)
        plsc.subcore_barrier()

        def load(t, buf, sem):
            return pltpu.async_copy(
                h2_hbm.at[pl.ds(base + t * PTILE, PTILE)], buf, sem)

        def wait(t, buf, sem):
            pltpu.make_async_copy(
                h2_hbm.at[pl.ds(base + t * PTILE, PTILE)], buf, sem).wait()

        load(0, ra, sa)

        @pl.loop(0, (NT - 1) // 2)
        def _(t):
            j0 = 2 * t
            load(j0 + 1, rb, sb)
            wait(j0, ra, sa)
            pltpu.sync_copy(ra, acc_sh.at[idx_v.at[j0]], add=True)
            load(j0 + 2, ra, sa)
            wait(j0 + 1, rb, sb)
            pltpu.sync_copy(rb, acc_sh.at[idx_v.at[j0 + 1]], add=True)

        wait(NT - 1, ra, sa)
        pltpu.sync_copy(ra, acc_sh.at[idx_v.at[NT - 1]], add=True)

        plsc.subcore_barrier()
        pltpu.sync_copy(acc_sh.at[pl.ds(sid * STRIPE, STRIPE)],
                        out_hbm.at[core, pl.ds(sid * STRIPE, STRIPE)])

    return k


_SC_AGG_EDGE = _make_sc_agg(CPW_EDGE, 2)
_SC_POOL = _make_sc_pool()
_SC_COUNT = _make_sc_count()


def _mm_scale_body(cnt_ref, x_ref, w_ref, y_ref, dinv_ref):
    deg = jnp.sum(cnt_ref[:, 0, :N], axis=0)[:, None] + 1.0
    dinv = lax.rsqrt(jnp.maximum(deg, 1.0))
    dinv_ref[...] = dinv
    y_ref[...] = dinv * jnp.dot(x_ref[...], w_ref[...],
                                preferred_element_type=jnp.float32)


def _mid_body(p_ref, y_ref, dinv_ref, b_ref, w_ref, o_ref):
    tot = p_ref[0, :N, :] + p_ref[1, :N, :] + y_ref[...]
    h = jnp.maximum(dinv_ref[...] * tot + b_ref[...], 0.0)
    o_ref[...] = dinv_ref[...] * jnp.dot(h, w_ref[...],
                                         preferred_element_type=jnp.float32)


def _final_body(p_ref, y_ref, dinv_ref, b_ref, o_ref):
    tot = p_ref[0, :N, :] + p_ref[1, :N, :] + y_ref[...]
    h2 = jnp.maximum(dinv_ref[...] * tot + b_ref[...], 0.0)
    o_ref[...] = jnp.concatenate(
        [h2, jnp.zeros((ACC_ROWS - N, D), jnp.float32)], axis=0)


def _div_body(sp_ref, cp_ref, z_ref):
    s = sp_ref[0, :N, :] + sp_ref[1, :N, :]
    cnt = jnp.sum(cp_ref[:, 1, :N], axis=0)[:, None]
    z_ref[...] = s / jnp.maximum(cnt, 1.0)


def _f32(shape):
    return jax.ShapeDtypeStruct(shape, jnp.float32)


def kernel(x, train_pos_edge_index, dict_node, W1, b1, W2, b2):
    src = train_pos_edge_index[0].astype(jnp.int32)
    dst = train_pos_edge_index[1].astype(jnp.int32)

    pad_e = jnp.arange(E_PAD - E, dtype=jnp.int32)
    src_p = jnp.concatenate(
        [src, pad_e % N]).reshape(NW, CPW_EDGE, CHUNK)
    dst_p = jnp.concatenate(
        [dst, N + pad_e % (ACC_ROWS - N)]).reshape(NW, CPW_EDGE, CHUNK)

    pad_p = jnp.arange(P_PAD - N, dtype=jnp.int32)
    pool_dst = jnp.concatenate(
        [dict_node.astype(jnp.int32),
         N + pad_p % (ACC_ROWS - N)]).reshape(NW, CPW_POOL, CHUNK)
    dpool = jnp.concatenate(
        [dict_node.astype(jnp.int32),
         N + jnp.arange(ACC_ROWS - N, dtype=jnp.int32)]).reshape(NW, NT, PTILE)

    counts = _SC_COUNT(dst_p, pool_dst).reshape(NW, 2, ACC_ROWS)
    y1, dinv = pl.pallas_call(
        _mm_scale_body,
        out_shape=(_f32((N, D)), _f32((N, 1))))(counts, x, W1)

    p1 = _SC_AGG_EDGE(y1, src_p, dst_p)
    y2 = pl.pallas_call(_mid_body, out_shape=_f32((N, D)))(
        p1, y1, dinv, b1.reshape(1, D), W2)

    p2 = _SC_AGG_EDGE(y2, src_p, dst_p)
    h2 = pl.pallas_call(_final_body, out_shape=_f32((ACC_ROWS, D)))(
        p2, y2, dinv, b2.reshape(1, D))

    sp = _SC_POOL(h2, dpool)
    z = pl.pallas_call(_div_body, out_shape=_f32((N, D)))(sp, counts)
    return z
